# Initial kernel scaffold; baseline (speedup 1.0000x reference)
import jax
import jax.numpy as jnp
from jax.experimental import pallas as pl


def kernel(x, edge_index, W1, b1, W2, b2):
    raise NotImplementedError("write your pallas kernel here")


# speedup vs baseline: 32.0431x; 32.0431x over previous
import functools

import jax
import jax.numpy as jnp
from jax import lax
from jax.experimental import pallas as pl
from jax.experimental.pallas import tpu as pltpu
from jax.experimental.pallas import tpu_sc as plsc

N = 10000
E = 320000
D = 128
H1 = 16
H2 = 8
NC = 2
NS = 16
NW = NC * NS
CH = 128
NCH = 80
EPW = NCH * CH
E_PAD = NW * EPW
NP = 10112
RPT = NP // NS
HN = 10240

_sc_mesh = plsc.VectorSubcoreMesh(core_axis_name="c", subcore_axis_name="s")
_sc_params = pltpu.CompilerParams(needs_layout_passes=False,
                                  use_tc_tiling_on_sc=False)


@functools.partial(
    pl.kernel,
    out_type=jax.ShapeDtypeStruct((NW, HN), jnp.float32),
    mesh=_sc_mesh,
    compiler_params=_sc_params,
    scratch_types=[
        pltpu.VMEM((EPW,), jnp.int32),
        pltpu.VMEM((HN,), jnp.float32),
    ],
)
def _deg_hist(dst_hbm, out_hbm, idx_v, hist_v):
    wid = lax.axis_index("s") * NC + lax.axis_index("c")
    pltpu.sync_copy(dst_hbm.at[pl.ds(wid * EPW, EPW)], idx_v)
    zeros = jnp.zeros((16,), jnp.float32)

    def zbody(i, carry):
        hist_v[pl.ds(i * 16, 16)] = zeros
        return carry

    lax.fori_loop(0, HN // 16, zbody, 0)
    ones = jnp.ones((16,), jnp.float32)

    def body(i, carry):
        vec = idx_v[pl.ds(i * 16, 16)]
        plsc.addupdate_scatter(hist_v, [vec], ones)
        return carry

    lax.fori_loop(0, EPW // 16, body, 0)
    pltpu.sync_copy(hist_v, out_hbm.at[wid])


@functools.partial(
    pl.kernel,
    out_type=jax.ShapeDtypeStruct((NC, NP, H1), jnp.float32),
    mesh=_sc_mesh,
    compiler_params=_sc_params,
    scratch_types=[
        pltpu.VMEM((NCH, CH), jnp.int32),
        pltpu.VMEM((NCH, CH), jnp.int32),
        pltpu.VMEM((CH, H1), jnp.float32),
        pltpu.VMEM((RPT, H1), jnp.float32),
        pltpu.VMEM_SHARED((NP, H1), jnp.float32),
        pltpu.SemaphoreType.DMA,
    ],
)
def _edge_scatter(g_hbm, src_hbm, dst_hbm, out_hbm,
                  src_v, dst_v, buf_v, z_v, acc_sh, sem):
    cid = lax.axis_index("c")
    sid = lax.axis_index("s")
    wid = sid * NC + cid
    zeros = jnp.zeros((16,), jnp.float32)

    def zbody(i, carry):
        z_v[i, :] = zeros
        return carry

    lax.fori_loop(0, RPT, zbody, 0)
    pltpu.sync_copy(z_v, acc_sh.at[pl.ds(sid * RPT, RPT)])
    pltpu.sync_copy(src_hbm.at[wid], src_v)
    pltpu.sync_copy(dst_hbm.at[wid], dst_v)
    plsc.subcore_barrier()

    def body(j, carry):
        pltpu.async_copy(g_hbm.at[src_v.at[j]], buf_v, sem).wait()
        pltpu.sync_copy(buf_v, acc_sh.at[dst_v.at[j]], add=True)
        return carry

    lax.fori_loop(0, NCH, body, 0)
    plsc.subcore_barrier()
    pltpu.sync_copy(acc_sh.at[pl.ds(sid * RPT, RPT)],
                    out_hbm.at[cid, pl.ds(sid * RPT, RPT)])


def _tc_a_body(hist_ref, x_ref, w1_ref, g1_ref, dinv_ref):
    deg = jnp.sum(hist_ref[...], axis=1, keepdims=True) + 1.0
    dinv = lax.rsqrt(deg[:N])
    p = jnp.dot(x_ref[...], w1_ref[...], preferred_element_type=jnp.float32)
    g1_ref[...] = p * dinv
    dinv_ref[...] = dinv


_tc_a = pl.pallas_call(
    _tc_a_body,
    out_shape=(jax.ShapeDtypeStruct((N, H1), jnp.float32),
               jax.ShapeDtypeStruct((N, 1), jnp.float32)),
)


def _tc_b_body(s_ref, g1_ref, dinv_ref, b1_ref, w2_ref, g2_ref):
    s = s_ref[0, :N, :] + s_ref[1, :N, :]
    dinv = dinv_ref[...]
    h1 = jnp.maximum(dinv * (s + g1_ref[...]) + b1_ref[...], 0.0)
    g2_ref[...] = jnp.dot(h1, w2_ref[...],
                          preferred_element_type=jnp.float32) * dinv


_tc_b = pl.pallas_call(
    _tc_b_body,
    out_shape=jax.ShapeDtypeStruct((N, H1), jnp.float32),
)


def _tc_c_body(s_ref, g2_ref, dinv_ref, b2_ref, out_ref):
    s = s_ref[0, :N, :] + s_ref[1, :N, :]
    t = dinv_ref[...] * (s + g2_ref[...])
    out_ref[...] = t[:, :H2] + b2_ref[...]


_tc_c = pl.pallas_call(
    _tc_c_body,
    out_shape=jax.ShapeDtypeStruct((N, H2), jnp.float32),
)


def kernel(x, edge_index, W1, b1, W2, b2):
    src = edge_index[0].astype(jnp.int32)
    dst = edge_index[1].astype(jnp.int32)
    pad = E_PAD - E
    src3 = jnp.concatenate([src, jnp.zeros((pad,), jnp.int32)]).reshape(NW, NCH, CH)
    dstp = jnp.concatenate([dst, jnp.full((pad,), N, jnp.int32)])
    dst3 = dstp.reshape(NW, NCH, CH)
    w2p = jnp.concatenate([W2, jnp.zeros((H1, H1 - H2), jnp.float32)], axis=1)

    hist = _deg_hist(dstp)
    g1, dinv = _tc_a(hist.T, x, W1)
    s1 = _edge_scatter(g1, src3, dst3)
    g2 = _tc_b(s1, g1, dinv, b1.reshape(1, H1), w2p)
    s2 = _edge_scatter(g2, src3, dst3)
    out = _tc_c(s2, g2, dinv, b2.reshape(1, H2))
    return out


# speedup vs baseline: 43.1912x; 1.3479x over previous
import functools

import jax
import jax.numpy as jnp
from jax import lax
from jax.experimental import pallas as pl
from jax.experimental.pallas import tpu as pltpu
from jax.experimental.pallas import tpu_sc as plsc

N = 10000
E = 320000
D = 128
H1 = 16
H2 = 8
NC = 2
NS = 16
NW = NC * NS
CH = 128
NCH = 80
EPW = NCH * CH
E_PAD = NW * EPW
NP = 10112
RPT = NP // NS
HN = 10240

_sc_mesh = plsc.VectorSubcoreMesh(core_axis_name="c", subcore_axis_name="s")
_sc_params = pltpu.CompilerParams(needs_layout_passes=False,
                                  use_tc_tiling_on_sc=False)


@functools.partial(
    pl.kernel,
    out_type=jax.ShapeDtypeStruct((NW, HN), jnp.float32),
    mesh=_sc_mesh,
    compiler_params=_sc_params,
    scratch_types=[
        pltpu.VMEM((EPW,), jnp.int32),
        pltpu.VMEM((HN,), jnp.float32),
    ],
)
def _deg_hist(dst_hbm, out_hbm, idx_v, hist_v):
    wid = lax.axis_index("s") * NC + lax.axis_index("c")
    pltpu.sync_copy(dst_hbm.at[pl.ds(wid * EPW, EPW)], idx_v)
    zeros = jnp.zeros((16,), jnp.float32)

    def zbody(i, carry):
        hist_v[pl.ds(i * 16, 16)] = zeros
        return carry

    lax.fori_loop(0, HN // 16, zbody, 0)
    ones = jnp.ones((16,), jnp.float32)

    def body(i, carry):
        vec = idx_v[pl.ds(i * 16, 16)]
        plsc.addupdate_scatter(hist_v, [vec], ones)
        return carry

    lax.fori_loop(0, EPW // 16, body, 0)
    pltpu.sync_copy(hist_v, out_hbm.at[wid])


@functools.partial(
    pl.kernel,
    out_type=jax.ShapeDtypeStruct((NC, NP, H1), jnp.float32),
    mesh=_sc_mesh,
    compiler_params=_sc_params,
    scratch_types=[
        pltpu.VMEM((NCH, CH), jnp.int32),
        pltpu.VMEM((NCH, CH), jnp.int32),
        pltpu.VMEM((4, CH, H1), jnp.float32),
        pltpu.VMEM((RPT, H1), jnp.float32),
        pltpu.VMEM_SHARED((NP, H1), jnp.float32),
        pltpu.SemaphoreType.DMA((4,)),
    ],
)
def _edge_scatter(g_hbm, src_hbm, dst_hbm, out_hbm,
                  src_v, dst_v, buf_v, z_v, acc_sh, gsem):
    cid = lax.axis_index("c")
    sid = lax.axis_index("s")
    wid = sid * NC + cid
    zeros = jnp.zeros((16,), jnp.float32)

    def zbody(i, carry):
        z_v[i, :] = zeros
        return carry

    lax.fori_loop(0, RPT, zbody, 0)
    pltpu.sync_copy(z_v, acc_sh.at[pl.ds(sid * RPT, RPT)])
    pltpu.sync_copy(src_hbm.at[wid], src_v)
    pltpu.sync_copy(dst_hbm.at[wid], dst_v)
    plsc.subcore_barrier()

    NBUF = 4
    LOOK = NBUF - 1
    gd = [None] * NCH
    for j in range(LOOK):
        gd[j] = pltpu.async_copy(
            g_hbm.at[src_v.at[j]], buf_v.at[j % NBUF], gsem.at[j % NBUF])
    for j in range(NCH):
        if j + LOOK < NCH:
            k = (j + LOOK) % NBUF
            gd[j + LOOK] = pltpu.async_copy(
                g_hbm.at[src_v.at[j + LOOK]], buf_v.at[k], gsem.at[k])
        gd[j].wait()
        pltpu.sync_copy(buf_v.at[j % NBUF], acc_sh.at[dst_v.at[j]], add=True)
    plsc.subcore_barrier()
    pltpu.sync_copy(acc_sh.at[pl.ds(sid * RPT, RPT)],
                    out_hbm.at[cid, pl.ds(sid * RPT, RPT)])


def _tc_a_body(hist_ref, x_ref, w1_ref, g1_ref, dinv_ref):
    deg = jnp.sum(hist_ref[...], axis=1, keepdims=True) + 1.0
    dinv = lax.rsqrt(deg[:N])
    p = jnp.dot(x_ref[...], w1_ref[...], preferred_element_type=jnp.float32)
    g1_ref[...] = p * dinv
    dinv_ref[...] = dinv


_tc_a = pl.pallas_call(
    _tc_a_body,
    out_shape=(jax.ShapeDtypeStruct((N, H1), jnp.float32),
               jax.ShapeDtypeStruct((N, 1), jnp.float32)),
)


def _tc_b_body(s_ref, g1_ref, dinv_ref, b1_ref, w2_ref, g2_ref):
    s = s_ref[0, :N, :] + s_ref[1, :N, :]
    dinv = dinv_ref[...]
    h1 = jnp.maximum(dinv * (s + g1_ref[...]) + b1_ref[...], 0.0)
    g2_ref[...] = jnp.dot(h1, w2_ref[...],
                          preferred_element_type=jnp.float32) * dinv


_tc_b = pl.pallas_call(
    _tc_b_body,
    out_shape=jax.ShapeDtypeStruct((N, H1), jnp.float32),
)


def _tc_c_body(s_ref, g2_ref, dinv_ref, b2_ref, out_ref):
    s = s_ref[0, :N, :] + s_ref[1, :N, :]
    t = dinv_ref[...] * (s + g2_ref[...])
    out_ref[...] = t[:, :H2] + b2_ref[...]


_tc_c = pl.pallas_call(
    _tc_c_body,
    out_shape=jax.ShapeDtypeStruct((N, H2), jnp.float32),
)


def kernel(x, edge_index, W1, b1, W2, b2):
    src = edge_index[0].astype(jnp.int32)
    dst = edge_index[1].astype(jnp.int32)
    pad = E_PAD - E
    src3 = jnp.concatenate([src, jnp.zeros((pad,), jnp.int32)]).reshape(NW, NCH, CH)
    dstp = jnp.concatenate([dst, jnp.full((pad,), N, jnp.int32)])
    dst3 = dstp.reshape(NW, NCH, CH)
    w2p = jnp.concatenate([W2, jnp.zeros((H1, H1 - H2), jnp.float32)], axis=1)

    hist = _deg_hist(dstp)
    g1, dinv = _tc_a(hist.T, x, W1)
    s1 = _edge_scatter(g1, src3, dst3)
    g2 = _tc_b(s1, g1, dinv, b1.reshape(1, H1), w2p)
    s2 = _edge_scatter(g2, src3, dst3)
    out = _tc_c(s2, g2, dinv, b2.reshape(1, H2))
    return out


# speedup vs baseline: 43.5323x; 1.0079x over previous
import functools

import jax
import jax.numpy as jnp
from jax import lax
from jax.experimental import pallas as pl
from jax.experimental.pallas import tpu as pltpu
from jax.experimental.pallas import tpu_sc as plsc

N = 10000
E = 320000
D = 128
H1 = 16
H2 = 8
NC = 2
NS = 16
NW = NC * NS
CH = 512
NCH = 20
EPW = NCH * CH
E_PAD = NW * EPW
NP = 10112
RPT = NP // NS
HN = 10240

_sc_mesh = plsc.VectorSubcoreMesh(core_axis_name="c", subcore_axis_name="s")
_sc_params = pltpu.CompilerParams(needs_layout_passes=False,
                                  use_tc_tiling_on_sc=False)


@functools.partial(
    pl.kernel,
    out_type=jax.ShapeDtypeStruct((NW, HN), jnp.float32),
    mesh=_sc_mesh,
    compiler_params=_sc_params,
    scratch_types=[
        pltpu.VMEM((EPW,), jnp.int32),
        pltpu.VMEM((HN,), jnp.float32),
    ],
)
def _deg_hist(dst_hbm, out_hbm, idx_v, hist_v):
    wid = lax.axis_index("s") * NC + lax.axis_index("c")
    pltpu.sync_copy(dst_hbm.at[pl.ds(wid * EPW, EPW)], idx_v)
    zeros = jnp.zeros((16,), jnp.float32)

    def zbody(i, carry):
        hist_v[pl.ds(i * 16, 16)] = zeros
        return carry

    lax.fori_loop(0, HN // 16, zbody, 0)
    ones = jnp.ones((16,), jnp.float32)

    def body(i, carry):
        vec = idx_v[pl.ds(i * 16, 16)]
        plsc.addupdate_scatter(hist_v, [vec], ones)
        return carry

    lax.fori_loop(0, EPW // 16, body, 0)
    pltpu.sync_copy(hist_v, out_hbm.at[wid])


@functools.partial(
    pl.kernel,
    out_type=jax.ShapeDtypeStruct((NC, NP, H1), jnp.float32),
    mesh=_sc_mesh,
    compiler_params=_sc_params,
    scratch_types=[
        pltpu.VMEM((NCH, CH), jnp.int32),
        pltpu.VMEM((NCH, CH), jnp.int32),
        pltpu.VMEM((4, CH, H1), jnp.float32),
        pltpu.VMEM((RPT, H1), jnp.float32),
        pltpu.VMEM_SHARED((NP, H1), jnp.float32),
        pltpu.SemaphoreType.DMA((4,)),
    ],
)
def _edge_scatter(g_hbm, src_hbm, dst_hbm, out_hbm,
                  src_v, dst_v, buf_v, z_v, acc_sh, gsem):
    cid = lax.axis_index("c")
    sid = lax.axis_index("s")
    wid = sid * NC + cid
    zeros = jnp.zeros((16,), jnp.float32)

    def zbody(i, carry):
        z_v[i, :] = zeros
        return carry

    lax.fori_loop(0, RPT, zbody, 0)
    pltpu.sync_copy(z_v, acc_sh.at[pl.ds(sid * RPT, RPT)])
    pltpu.sync_copy(src_hbm.at[wid], src_v)
    pltpu.sync_copy(dst_hbm.at[wid], dst_v)
    plsc.subcore_barrier()

    NBUF = 4
    LOOK = NBUF - 1
    gd = [None] * NCH
    for j in range(LOOK):
        gd[j] = pltpu.async_copy(
            g_hbm.at[src_v.at[j]], buf_v.at[j % NBUF], gsem.at[j % NBUF])
    for j in range(NCH):
        if j + LOOK < NCH:
            k = (j + LOOK) % NBUF
            gd[j + LOOK] = pltpu.async_copy(
                g_hbm.at[src_v.at[j + LOOK]], buf_v.at[k], gsem.at[k])
        gd[j].wait()
        pltpu.sync_copy(buf_v.at[j % NBUF], acc_sh.at[dst_v.at[j]], add=True)
    plsc.subcore_barrier()
    pltpu.sync_copy(acc_sh.at[pl.ds(sid * RPT, RPT)],
                    out_hbm.at[cid, pl.ds(sid * RPT, RPT)])


def _tc_a_body(hist_ref, x_ref, w1_ref, g1_ref, dinv_ref):
    deg = jnp.sum(hist_ref[...], axis=1, keepdims=True) + 1.0
    dinv = lax.rsqrt(deg[:N])
    p = jnp.dot(x_ref[...], w1_ref[...], preferred_element_type=jnp.float32)
    g1_ref[...] = p * dinv
    dinv_ref[...] = dinv


_tc_a = pl.pallas_call(
    _tc_a_body,
    out_shape=(jax.ShapeDtypeStruct((N, H1), jnp.float32),
               jax.ShapeDtypeStruct((N, 1), jnp.float32)),
)


def _tc_b_body(s_ref, g1_ref, dinv_ref, b1_ref, w2_ref, g2_ref):
    s = s_ref[0, :N, :] + s_ref[1, :N, :]
    dinv = dinv_ref[...]
    h1 = jnp.maximum(dinv * (s + g1_ref[...]) + b1_ref[...], 0.0)
    g2_ref[...] = jnp.dot(h1, w2_ref[...],
                          preferred_element_type=jnp.float32) * dinv


_tc_b = pl.pallas_call(
    _tc_b_body,
    out_shape=jax.ShapeDtypeStruct((N, H1), jnp.float32),
)


def _tc_c_body(s_ref, g2_ref, dinv_ref, b2_ref, out_ref):
    s = s_ref[0, :N, :] + s_ref[1, :N, :]
    t = dinv_ref[...] * (s + g2_ref[...])
    out_ref[...] = t[:, :H2] + b2_ref[...]


_tc_c = pl.pallas_call(
    _tc_c_body,
    out_shape=jax.ShapeDtypeStruct((N, H2), jnp.float32),
)


def kernel(x, edge_index, W1, b1, W2, b2):
    src = edge_index[0].astype(jnp.int32)
    dst = edge_index[1].astype(jnp.int32)
    pad = E_PAD - E
    src3 = jnp.concatenate([src, jnp.zeros((pad,), jnp.int32)]).reshape(NW, NCH, CH)
    dstp = jnp.concatenate([dst, jnp.full((pad,), N, jnp.int32)])
    dst3 = dstp.reshape(NW, NCH, CH)
    w2p = jnp.concatenate([W2, jnp.zeros((H1, H1 - H2), jnp.float32)], axis=1)

    hist = _deg_hist(dstp)
    g1, dinv = _tc_a(hist.T, x, W1)
    s1 = _edge_scatter(g1, src3, dst3)
    g2 = _tc_b(s1, g1, dinv, b1.reshape(1, H1), w2p)
    s2 = _edge_scatter(g2, src3, dst3)
    out = _tc_c(s2, g2, dinv, b2.reshape(1, H2))
    return out


# speedup vs baseline: 43.7582x; 1.0052x over previous
import functools

import jax
import jax.numpy as jnp
from jax import lax
from jax.experimental import pallas as pl
from jax.experimental.pallas import tpu as pltpu
from jax.experimental.pallas import tpu_sc as plsc

N = 10000
E = 320000
D = 128
H1 = 16
H2 = 8
NC = 2
NS = 16
NW = NC * NS
CH = 512
NCH = 20
EPW = NCH * CH
E_PAD = NW * EPW
NP = 10112
RPT = NP // NS
HN = 10240

_sc_mesh = plsc.VectorSubcoreMesh(core_axis_name="c", subcore_axis_name="s")
_sc_params = pltpu.CompilerParams(needs_layout_passes=False,
                                  use_tc_tiling_on_sc=False)


@functools.partial(
    pl.kernel,
    out_type=jax.ShapeDtypeStruct((NW, HN), jnp.float32),
    mesh=_sc_mesh,
    compiler_params=_sc_params,
    scratch_types=[
        pltpu.VMEM((EPW,), jnp.int32),
        pltpu.VMEM((HN,), jnp.float32),
    ],
)
def _deg_hist(dst_hbm, out_hbm, idx_v, hist_v):
    wid = lax.axis_index("s") * NC + lax.axis_index("c")
    pltpu.sync_copy(dst_hbm.at[pl.ds(wid * EPW, EPW)], idx_v)
    zeros = jnp.zeros((16,), jnp.float32)

    def zbody(i, carry):
        hist_v[pl.ds(i * 16, 16)] = zeros
        return carry

    lax.fori_loop(0, HN // 16, zbody, 0)
    ones = jnp.ones((16,), jnp.float32)

    def body(i, carry):
        vec = idx_v[pl.ds(i * 16, 16)]
        plsc.addupdate_scatter(hist_v, [vec], ones)
        return carry

    lax.fori_loop(0, EPW // 16, body, 0)
    pltpu.sync_copy(hist_v, out_hbm.at[wid])


@functools.partial(
    pl.kernel,
    out_type=jax.ShapeDtypeStruct((NC, NP, H1), jnp.float32),
    mesh=_sc_mesh,
    compiler_params=_sc_params,
    scratch_types=[
        pltpu.VMEM((NCH, CH), jnp.int32),
        pltpu.VMEM((NCH, CH), jnp.int32),
        pltpu.VMEM((4, CH, H1), jnp.float32),
        pltpu.VMEM((RPT, H1), jnp.float32),
        pltpu.VMEM_SHARED((NP, H1), jnp.float32),
        pltpu.SemaphoreType.DMA((4,)),
        pltpu.SemaphoreType.DMA((2,)),
    ],
)
def _edge_scatter(g_hbm, src_hbm, dst_hbm, out_hbm,
                  src_v, dst_v, buf_v, z_v, acc_sh, gsem, ssem):
    cid = lax.axis_index("c")
    sid = lax.axis_index("s")
    wid = sid * NC + cid
    zeros = jnp.zeros((16,), jnp.float32)

    def zbody(i, carry):
        z_v[i, :] = zeros
        return carry

    lax.fori_loop(0, RPT, zbody, 0)
    pltpu.sync_copy(z_v, acc_sh.at[pl.ds(sid * RPT, RPT)])
    pltpu.sync_copy(src_hbm.at[wid], src_v)
    pltpu.sync_copy(dst_hbm.at[wid], dst_v)
    plsc.subcore_barrier()

    NBUF = 4
    LOOK = 2
    gd = [None] * NCH
    sd = [None] * NCH
    for j in range(LOOK):
        gd[j] = pltpu.async_copy(
            g_hbm.at[src_v.at[j]], buf_v.at[j % NBUF], gsem.at[j % NBUF])
    for j in range(NCH):
        if j + LOOK < NCH:
            k = (j + LOOK) % NBUF
            if j + LOOK >= NBUF:
                sd[j + LOOK - NBUF].wait()
            gd[j + LOOK] = pltpu.async_copy(
                g_hbm.at[src_v.at[j + LOOK]], buf_v.at[k], gsem.at[k])
        gd[j].wait()
        sd[j] = pltpu.async_copy(
            buf_v.at[j % NBUF], acc_sh.at[dst_v.at[j]], ssem.at[j % 2],
            add=True)
    for i in range(NCH - NBUF, NCH):
        sd[i].wait()
    plsc.subcore_barrier()
    pltpu.sync_copy(acc_sh.at[pl.ds(sid * RPT, RPT)],
                    out_hbm.at[cid, pl.ds(sid * RPT, RPT)])


def _tc_a_body(hist_ref, x_ref, w1_ref, g1_ref, dinv_ref):
    deg = jnp.sum(hist_ref[...], axis=1, keepdims=True) + 1.0
    dinv = lax.rsqrt(deg[:N])
    p = jnp.dot(x_ref[...], w1_ref[...], preferred_element_type=jnp.float32)
    g1_ref[...] = p * dinv
    dinv_ref[...] = dinv


_tc_a = pl.pallas_call(
    _tc_a_body,
    out_shape=(jax.ShapeDtypeStruct((N, H1), jnp.float32),
               jax.ShapeDtypeStruct((N, 1), jnp.float32)),
)


def _tc_b_body(s_ref, g1_ref, dinv_ref, b1_ref, w2_ref, g2_ref):
    s = s_ref[0, :N, :] + s_ref[1, :N, :]
    dinv = dinv_ref[...]
    h1 = jnp.maximum(dinv * (s + g1_ref[...]) + b1_ref[...], 0.0)
    g2_ref[...] = jnp.dot(h1, w2_ref[...],
                          preferred_element_type=jnp.float32) * dinv


_tc_b = pl.pallas_call(
    _tc_b_body,
    out_shape=jax.ShapeDtypeStruct((N, H1), jnp.float32),
)


def _tc_c_body(s_ref, g2_ref, dinv_ref, b2_ref, out_ref):
    s = s_ref[0, :N, :] + s_ref[1, :N, :]
    t = dinv_ref[...] * (s + g2_ref[...])
    out_ref[...] = t[:, :H2] + b2_ref[...]


_tc_c = pl.pallas_call(
    _tc_c_body,
    out_shape=jax.ShapeDtypeStruct((N, H2), jnp.float32),
)


def kernel(x, edge_index, W1, b1, W2, b2):
    src = edge_index[0].astype(jnp.int32)
    dst = edge_index[1].astype(jnp.int32)
    pad = E_PAD - E
    src3 = jnp.concatenate([src, jnp.zeros((pad,), jnp.int32)]).reshape(NW, NCH, CH)
    dstp = jnp.concatenate([dst, jnp.full((pad,), N, jnp.int32)])
    dst3 = dstp.reshape(NW, NCH, CH)
    w2p = jnp.concatenate([W2, jnp.zeros((H1, H1 - H2), jnp.float32)], axis=1)

    hist = _deg_hist(dstp)
    g1, dinv = _tc_a(hist.T, x, W1)
    s1 = _edge_scatter(g1, src3, dst3)
    g2 = _tc_b(s1, g1, dinv, b1.reshape(1, H1), w2p)
    s2 = _edge_scatter(g2, src3, dst3)
    out = _tc_c(s2, g2, dinv, b2.reshape(1, H2))
    return out


# speedup vs baseline: 95.0728x; 2.1727x over previous
import functools

import jax
import jax.numpy as jnp
from jax import lax
from jax.experimental import pallas as pl
from jax.experimental.pallas import tpu as pltpu
from jax.experimental.pallas import tpu_sc as plsc

N = 10000
E = 320000
H1 = 16
H2 = 8
NC = 2
NS = 16
NW = NC * NS
NP = 10240
RPT = NP // NS
EPT = E // NW
CH = 512
NFC = EPT // CH
TL = EPT - NFC * CH
EPH = E // NS
BPS = NP // NW

_sc_mesh = plsc.VectorSubcoreMesh(core_axis_name="c", subcore_axis_name="s")
_sc_params = pltpu.CompilerParams(needs_layout_passes=False,
                                  use_tc_tiling_on_sc=False)


@functools.partial(
    pl.kernel,
    out_type=jax.ShapeDtypeStruct((NP * H1,), jnp.float32),
    mesh=_sc_mesh,
    compiler_params=_sc_params,
    scratch_types=[
        pltpu.VMEM((EPH,), jnp.int32),
        pltpu.VMEM((NP,), jnp.float32),
        pltpu.VMEM((NS, BPS), jnp.float32),
        pltpu.VMEM((BPS * H1,), jnp.float32),
        pltpu.VMEM_SHARED((NS, NP), jnp.float32),
    ],
)
def _deg_dinv(ei_hbm, out_hbm, idx_v, hist_v, red_v, d2_v, stage_sh):
    cid = lax.axis_index("c")
    sid = lax.axis_index("s")
    pltpu.sync_copy(ei_hbm.at[1, pl.ds(sid * EPH, EPH)], idx_v)
    zeros = jnp.zeros((16,), jnp.float32)

    def zbody(i, carry):
        hist_v[pl.ds(i * 16, 16)] = zeros
        return carry

    lax.fori_loop(0, NP // 16, zbody, 0)
    ones = jnp.ones((16,), jnp.float32)

    def hbody(i, carry):
        vec = idx_v[pl.ds(i * 16, 16)]
        plsc.addupdate_scatter(hist_v, [vec], ones)
        return carry

    lax.fori_loop(0, EPH // 16, hbody, 0)
    pltpu.sync_copy(hist_v, stage_sh.at[sid])
    plsc.subcore_barrier()
    g = sid * NC + cid
    for k in range(NS):
        pltpu.sync_copy(stage_sh.at[k, pl.ds(g * BPS, BPS)], red_v.at[k])
    lanes = jnp.arange(16, dtype=jnp.int32)
    half = jnp.full((16,), 0.5, jnp.float32)
    threehalf = jnp.full((16,), 1.5, jnp.float32)
    magic = jnp.full((16,), 0x5F3759DF, jnp.int32)

    def dbody(i, carry):
        acc = jnp.ones((16,), jnp.float32)
        for k in range(NS):
            acc = acc + red_v[k, pl.ds(i * 16, 16)]
        yi = magic - lax.shift_right_logical(plsc.bitcast(acc, jnp.int32), 1)
        y = plsc.bitcast(yi, jnp.float32)
        hx = half * acc
        for _ in range(3):
            y = y * (threehalf - hx * y * y)
        base = i * (16 * H1) + lanes * H1
        for j in range(H1):
            plsc.store_scatter(d2_v, [base + j], y)
        return carry

    lax.fori_loop(0, BPS // 16, dbody, 0)
    pltpu.sync_copy(d2_v, out_hbm.at[pl.ds(g * (BPS * H1), BPS * H1)])


@functools.partial(
    pl.kernel,
    out_type=jax.ShapeDtypeStruct((NC, NP, H1), jnp.float32),
    mesh=_sc_mesh,
    compiler_params=_sc_params,
    scratch_types=[
        pltpu.VMEM((4, CH), jnp.int32),
        pltpu.VMEM((4, CH), jnp.int32),
        pltpu.VMEM((TL,), jnp.int32),
        pltpu.VMEM((TL,), jnp.int32),
        pltpu.VMEM((4, CH, H1), jnp.float32),
        pltpu.VMEM((TL, H1), jnp.float32),
        pltpu.VMEM((RPT, H1), jnp.float32),
        pltpu.VMEM_SHARED((NP, H1), jnp.float32),
        pltpu.SemaphoreType.DMA((4,)),
        pltpu.SemaphoreType.DMA((4,)),
        pltpu.SemaphoreType.DMA((2,)),
        pltpu.SemaphoreType.DMA,
    ],
)
def _edge_scatter(g_hbm, ei_hbm, out_hbm, sidx_v, didx_v, tsidx_v, tdidx_v,
                  buf_v, tbuf_v, z_v, acc_sh, isem, gsem, ssem, tsem):
    cid = lax.axis_index("c")
    sid = lax.axis_index("s")
    wid = sid * NC + cid
    base = wid * EPT
    zeros = jnp.zeros((16,), jnp.float32)

    def zbody(i, carry):
        z_v[i, :] = zeros
        return carry

    lax.fori_loop(0, RPT, zbody, 0)
    pltpu.sync_copy(z_v, acc_sh.at[pl.ds(sid * RPT, RPT)])
    tid = pltpu.async_copy(ei_hbm.at[0, pl.ds(base + NFC * CH, TL)],
                           tsidx_v, tsem)
    tdd = pltpu.async_copy(ei_hbm.at[1, pl.ds(base + NFC * CH, TL)],
                           tdidx_v, tsem)
    plsc.subcore_barrier()

    idd = [None] * NFC
    gd = [None] * NFC
    sd = [None] * NFC

    def fire_idx(j):
        k = j % 4
        idd[j] = (
            pltpu.async_copy(ei_hbm.at[0, pl.ds(base + j * CH, CH)],
                             sidx_v.at[k], isem.at[k]),
            pltpu.async_copy(ei_hbm.at[1, pl.ds(base + j * CH, CH)],
                             didx_v.at[k], isem.at[k]),
        )

    def fire_gather(j):
        k = j % 4
        idd[j][0].wait()
        idd[j][1].wait()
        gd[j] = pltpu.async_copy(g_hbm.at[sidx_v.at[k]], buf_v.at[k],
                                 gsem.at[k])

    def fire_scatter(j):
        k = j % 4
        gd[j].wait()
        sd[j] = pltpu.async_copy(buf_v.at[k], acc_sh.at[didx_v.at[k]],
                                 ssem.at[j % 2], add=True)

    fire_idx(0)
    fire_idx(1)
    fire_gather(0)
    for j in range(NFC):
        if j + 2 < NFC:
            if j + 2 >= 4:
                sd[j - 2].wait()
            fire_idx(j + 2)
        if j + 1 < NFC:
            fire_gather(j + 1)
        fire_scatter(j)
    for j in range(NFC - 4, NFC):
        sd[j].wait()
    tid.wait()
    tdd.wait()
    pltpu.async_copy(g_hbm.at[tsidx_v], tbuf_v, tsem).wait()
    pltpu.sync_copy(tbuf_v, acc_sh.at[tdidx_v], add=True)
    plsc.subcore_barrier()
    pltpu.sync_copy(acc_sh.at[pl.ds(sid * RPT, RPT)],
                    out_hbm.at[cid, pl.ds(sid * RPT, RPT)])


NPK = NP // 8
NK = N // 8


def _tc_a_body(xg_ref, w1e_ref, d2_ref, g1_ref):
    p = jnp.dot(xg_ref[...], w1e_ref[...], preferred_element_type=jnp.float32)
    g1_ref[...] = p * d2_ref[:NK, :]


_tc_a = pl.pallas_call(
    _tc_a_body,
    out_shape=jax.ShapeDtypeStruct((NK, 128), jnp.float32),
)


def _tc_b_body(s_ref, g1_ref, d2_ref, b1_ref, w2_ref, g2_ref):
    s = s_ref[0, :NK, :] + s_ref[1, :NK, :]
    d2 = d2_ref[:NK, :]
    h1 = jnp.maximum(d2 * (s + g1_ref[...]) + b1_ref[...], 0.0)
    g2_ref[...] = jnp.dot(h1, w2_ref[...],
                          preferred_element_type=jnp.float32) * d2


_tc_b = pl.pallas_call(
    _tc_b_body,
    out_shape=jax.ShapeDtypeStruct((NK, 128), jnp.float32),
)


def _tc_c_body(s_ref, g2_ref, d2_ref, b2_ref, out_ref):
    s = s_ref[0, :NK, :] + s_ref[1, :NK, :]
    out_ref[...] = d2_ref[:NK, :] * s + g2_ref[...] * d2_ref[:NK, :] \
        + b2_ref[...]


_tc_c = pl.pallas_call(
    _tc_c_body,
    out_shape=jax.ShapeDtypeStruct((NK, 128), jnp.float32),
)


def kernel(x, edge_index, W1, b1, W2, b2):
    ei = edge_index.astype(jnp.int32)
    eye8 = jnp.eye(8, dtype=jnp.float32)
    w1e = jnp.kron(eye8, W1)
    w2bd = jnp.kron(eye8, jnp.pad(W2, ((0, 0), (0, H1 - H2))))
    b1t = jnp.tile(b1, 8)[None, :]
    b2t = jnp.tile(jnp.pad(b2, (0, H1 - H2)), 8)[None, :]

    d2p = _deg_dinv(ei).reshape(NPK, 128)
    xg = x.reshape(NK, 8 * 128)
    g1p = _tc_a(xg, w1e, d2p)
    s1 = _edge_scatter(g1p.reshape(N, H1), ei)
    g2p = _tc_b(s1.reshape(NC, NPK, 128), g1p, d2p, b1t, w2bd)
    s2 = _edge_scatter(g2p.reshape(N, H1), ei)
    outp = _tc_c(s2.reshape(NC, NPK, 128), g2p, d2p, b2t)
    return outp.reshape(N, H1)[:, :H2]


# speedup vs baseline: 97.7581x; 1.0282x over previous
import functools

import jax
import jax.numpy as jnp
from jax import lax
from jax.experimental import pallas as pl
from jax.experimental.pallas import tpu as pltpu
from jax.experimental.pallas import tpu_sc as plsc

N = 10000
E = 320000
H1 = 16
H2 = 8
NC = 2
NS = 16
NW = NC * NS
NP = 10240
RPT = NP // NS
EPT = E // NW
CH = 1000
NFC = EPT // CH
EPH = E // NS
BPS = NP // NW

_sc_mesh = plsc.VectorSubcoreMesh(core_axis_name="c", subcore_axis_name="s")
_sc_params = pltpu.CompilerParams(needs_layout_passes=False,
                                  use_tc_tiling_on_sc=False)


@functools.partial(
    pl.kernel,
    out_type=jax.ShapeDtypeStruct((NP * H1,), jnp.float32),
    mesh=_sc_mesh,
    compiler_params=_sc_params,
    scratch_types=[
        pltpu.VMEM((EPH,), jnp.int32),
        pltpu.VMEM((NP,), jnp.float32),
        pltpu.VMEM((NS, BPS), jnp.float32),
        pltpu.VMEM((BPS * H1,), jnp.float32),
        pltpu.VMEM_SHARED((NS, NP), jnp.float32),
    ],
)
def _deg_dinv(ei_hbm, out_hbm, idx_v, hist_v, red_v, d2_v, stage_sh):
    cid = lax.axis_index("c")
    sid = lax.axis_index("s")
    pltpu.sync_copy(ei_hbm.at[1, pl.ds(sid * EPH, EPH)], idx_v)
    zeros = jnp.zeros((16,), jnp.float32)

    def zbody(i, carry):
        hist_v[pl.ds(i * 16, 16)] = zeros
        return carry

    lax.fori_loop(0, NP // 16, zbody, 0)
    ones = jnp.ones((16,), jnp.float32)

    def hbody(i, carry):
        for u in range(5):
            vec = idx_v[pl.ds(i * 80 + u * 16, 16)]
            plsc.addupdate_scatter(hist_v, [vec], ones)
        return carry

    lax.fori_loop(0, EPH // 80, hbody, 0)
    pltpu.sync_copy(hist_v, stage_sh.at[sid])
    plsc.subcore_barrier()
    g = sid * NC + cid
    for k in range(NS):
        pltpu.sync_copy(stage_sh.at[k, pl.ds(g * BPS, BPS)], red_v.at[k])
    lanes = jnp.arange(16, dtype=jnp.int32)
    half = jnp.full((16,), 0.5, jnp.float32)
    threehalf = jnp.full((16,), 1.5, jnp.float32)
    magic = jnp.full((16,), 0x5F3759DF, jnp.int32)

    def dbody(i, carry):
        acc = jnp.ones((16,), jnp.float32)
        for k in range(NS):
            acc = acc + red_v[k, pl.ds(i * 16, 16)]
        yi = magic - lax.shift_right_logical(plsc.bitcast(acc, jnp.int32), 1)
        y = plsc.bitcast(yi, jnp.float32)
        hx = half * acc
        for _ in range(3):
            y = y * (threehalf - hx * y * y)
        base = i * (16 * H1) + lanes * H1
        for j in range(H1):
            plsc.store_scatter(d2_v, [base + j], y)
        return carry

    lax.fori_loop(0, BPS // 16, dbody, 0)
    pltpu.sync_copy(d2_v, out_hbm.at[pl.ds(g * (BPS * H1), BPS * H1)])


@functools.partial(
    pl.kernel,
    out_type=jax.ShapeDtypeStruct((NC, NP, H1), jnp.float32),
    mesh=_sc_mesh,
    compiler_params=_sc_params,
    scratch_types=[
        pltpu.VMEM((4, CH), jnp.int32),
        pltpu.VMEM((4, CH), jnp.int32),
        pltpu.VMEM((4, CH, H1), jnp.float32),
        pltpu.VMEM((RPT, H1), jnp.float32),
        pltpu.VMEM_SHARED((NP, H1), jnp.float32),
        pltpu.SemaphoreType.DMA((4,)),
        pltpu.SemaphoreType.DMA((4,)),
        pltpu.SemaphoreType.DMA((2,)),
    ],
)
def _edge_scatter(g_hbm, ei_hbm, out_hbm, sidx_v, didx_v,
                  buf_v, z_v, acc_sh, isem, gsem, ssem):
    cid = lax.axis_index("c")
    sid = lax.axis_index("s")
    wid = sid * NC + cid
    base = wid * EPT
    zeros = jnp.zeros((16,), jnp.float32)

    def zbody(i, carry):
        z_v[i, :] = zeros
        return carry

    lax.fori_loop(0, RPT, zbody, 0)
    pltpu.sync_copy(z_v, acc_sh.at[pl.ds(sid * RPT, RPT)])
    plsc.subcore_barrier()

    idd = [None] * NFC
    gd = [None] * NFC
    sd = [None] * NFC

    def fire_idx(j):
        k = j % 4
        idd[j] = (
            pltpu.async_copy(ei_hbm.at[0, pl.ds(base + j * CH, CH)],
                             sidx_v.at[k], isem.at[k]),
            pltpu.async_copy(ei_hbm.at[1, pl.ds(base + j * CH, CH)],
                             didx_v.at[k], isem.at[k]),
        )

    def fire_gather(j):
        k = j % 4
        idd[j][0].wait()
        idd[j][1].wait()
        gd[j] = pltpu.async_copy(g_hbm.at[sidx_v.at[k]], buf_v.at[k],
                                 gsem.at[k])

    def fire_scatter(j):
        k = j % 4
        gd[j].wait()
        sd[j] = pltpu.async_copy(buf_v.at[k], acc_sh.at[didx_v.at[k]],
                                 ssem.at[j % 2], add=True)

    fire_idx(0)
    fire_idx(1)
    fire_gather(0)
    for j in range(NFC):
        if j + 2 < NFC:
            if j + 2 >= 4:
                sd[j - 2].wait()
            fire_idx(j + 2)
        if j + 1 < NFC:
            fire_gather(j + 1)
        fire_scatter(j)
    for j in range(NFC - 4, NFC):
        sd[j].wait()
    plsc.subcore_barrier()
    pltpu.sync_copy(acc_sh.at[pl.ds(sid * RPT, RPT)],
                    out_hbm.at[cid, pl.ds(sid * RPT, RPT)])


NPK = NP // 8
NK = N // 8


def _tc_a_body(xg_ref, w1e_ref, d2_ref, g1_ref):
    p = jnp.dot(xg_ref[...], w1e_ref[...], preferred_element_type=jnp.float32)
    g1_ref[...] = p * d2_ref[:NK, :]


_tc_a = pl.pallas_call(
    _tc_a_body,
    out_shape=jax.ShapeDtypeStruct((NK, 128), jnp.float32),
)


def _tc_b_body(s_ref, g1_ref, d2_ref, b1_ref, w2_ref, g2_ref):
    s = s_ref[0, :NK, :] + s_ref[1, :NK, :]
    d2 = d2_ref[:NK, :]
    h1 = jnp.maximum(d2 * (s + g1_ref[...]) + b1_ref[...], 0.0)
    g2_ref[...] = jnp.dot(h1, w2_ref[...],
                          preferred_element_type=jnp.float32) * d2


_tc_b = pl.pallas_call(
    _tc_b_body,
    out_shape=jax.ShapeDtypeStruct((NK, 128), jnp.float32),
)


def _tc_c_body(s_ref, g2_ref, d2_ref, b2_ref, out_ref):
    s = s_ref[0, :NK, :] + s_ref[1, :NK, :]
    out_ref[...] = d2_ref[:NK, :] * s + g2_ref[...] * d2_ref[:NK, :] \
        + b2_ref[...]


_tc_c = pl.pallas_call(
    _tc_c_body,
    out_shape=jax.ShapeDtypeStruct((NK, 128), jnp.float32),
)


def kernel(x, edge_index, W1, b1, W2, b2):
    ei = edge_index.astype(jnp.int32)
    eye8 = jnp.eye(8, dtype=jnp.float32)
    w1e = jnp.kron(eye8, W1)
    w2bd = jnp.kron(eye8, jnp.pad(W2, ((0, 0), (0, H1 - H2))))
    b1t = jnp.tile(b1, 8)[None, :]
    b2t = jnp.tile(jnp.pad(b2, (0, H1 - H2)), 8)[None, :]

    d2p = _deg_dinv(ei).reshape(NPK, 128)
    xg = x.reshape(NK, 8 * 128)
    g1p = _tc_a(xg, w1e, d2p)
    s1 = _edge_scatter(g1p.reshape(N, H1), ei)
    g2p = _tc_b(s1.reshape(NC, NPK, 128), g1p, d2p, b1t, w2bd)
    s2 = _edge_scatter(g2p.reshape(N, H1), ei)
    outp = _tc_c(s2.reshape(NC, NPK, 128), g2p, d2p, b2t)
    return outp.reshape(N, H1)[:, :H2]


# speedup vs baseline: 106.5687x; 1.0901x over previous
import functools

import jax
import jax.numpy as jnp
from jax import lax
from jax.experimental import pallas as pl
from jax.experimental.pallas import tpu as pltpu
from jax.experimental.pallas import tpu_sc as plsc

N = 10000
E = 320000
H1 = 16
H2 = 8
NC = 2
NS = 16
NW = NC * NS
NP = 10240
RPT = NP // NS
EPT = E // NW
CH = 1000
NFC = EPT // CH
EPH = E // NS
BPS = NP // NW

_sc_mesh = plsc.VectorSubcoreMesh(core_axis_name="c", subcore_axis_name="s")
_sc_params = pltpu.CompilerParams(needs_layout_passes=False,
                                  use_tc_tiling_on_sc=False)


@functools.partial(
    pl.kernel,
    out_type=jax.ShapeDtypeStruct((NP * H1,), jnp.float32),
    mesh=_sc_mesh,
    compiler_params=_sc_params,
    scratch_types=[
        pltpu.VMEM((EPH,), jnp.int32),
        pltpu.VMEM((NP,), jnp.float32),
        pltpu.VMEM((NS, BPS), jnp.float32),
        pltpu.VMEM((BPS * H1,), jnp.float32),
        pltpu.VMEM_SHARED((NS, NP), jnp.float32),
    ],
)
def _deg_dinv(ei_hbm, out_hbm, idx_v, hist_v, red_v, d2_v, stage_sh):
    cid = lax.axis_index("c")
    sid = lax.axis_index("s")
    pltpu.sync_copy(ei_hbm.at[1, pl.ds(sid * EPH, EPH)], idx_v)
    zeros = jnp.zeros((16,), jnp.float32)

    def zbody(i, carry):
        for u in range(8):
            hist_v[pl.ds(i * 128 + u * 16, 16)] = zeros
        return carry

    lax.fori_loop(0, NP // 128, zbody, 0)
    ones = jnp.ones((16,), jnp.float32)

    def hbody(i, carry):
        for u in range(5):
            vec = idx_v[pl.ds(i * 80 + u * 16, 16)]
            plsc.addupdate_scatter(hist_v, [vec], ones)
        return carry

    lax.fori_loop(0, EPH // 80, hbody, 0)
    pltpu.sync_copy(hist_v, stage_sh.at[sid])
    plsc.subcore_barrier()
    g = sid * NC + cid
    for k in range(NS):
        pltpu.sync_copy(stage_sh.at[k, pl.ds(g * BPS, BPS)], red_v.at[k])
    lanes = jnp.arange(16, dtype=jnp.int32)
    half = jnp.full((16,), 0.5, jnp.float32)
    threehalf = jnp.full((16,), 1.5, jnp.float32)
    magic = jnp.full((16,), 0x5F3759DF, jnp.int32)

    def dbody(i, carry):
        acc = jnp.ones((16,), jnp.float32)
        for k in range(NS):
            acc = acc + red_v[k, pl.ds(i * 16, 16)]
        yi = magic - lax.shift_right_logical(plsc.bitcast(acc, jnp.int32), 1)
        y = plsc.bitcast(yi, jnp.float32)
        hx = half * acc
        for _ in range(3):
            y = y * (threehalf - hx * y * y)
        base = i * (16 * H1) + lanes * H1
        for j in range(H1):
            plsc.store_scatter(d2_v, [base + j], y)
        return carry

    lax.fori_loop(0, BPS // 16, dbody, 0)
    pltpu.sync_copy(d2_v, out_hbm.at[pl.ds(g * (BPS * H1), BPS * H1)])


@functools.partial(
    pl.kernel,
    out_type=jax.ShapeDtypeStruct((NC, NP, H1), jnp.float32),
    mesh=_sc_mesh,
    compiler_params=_sc_params,
    scratch_types=[
        pltpu.VMEM((4, CH), jnp.int32),
        pltpu.VMEM((4, CH), jnp.int32),
        pltpu.VMEM((4, CH, H1), jnp.float32),
        pltpu.VMEM((RPT, H1), jnp.float32),
        pltpu.VMEM_SHARED((NP, H1), jnp.float32),
        pltpu.SemaphoreType.DMA((4,)),
        pltpu.SemaphoreType.DMA((4,)),
        pltpu.SemaphoreType.DMA((2,)),
    ],
)
def _edge_scatter(g_hbm, ei_hbm, out_hbm, sidx_v, didx_v,
                  buf_v, z_v, acc_sh, isem, gsem, ssem):
    cid = lax.axis_index("c")
    sid = lax.axis_index("s")
    wid = sid * NC + cid
    base = wid * EPT
    zeros = jnp.zeros((16,), jnp.float32)

    def zbody(i, carry):
        for u in range(8):
            z_v[i * 8 + u, :] = zeros
        return carry

    lax.fori_loop(0, RPT // 8, zbody, 0)

    idd = [None] * NFC
    gd = [None] * NFC
    sd = [None] * NFC

    def fire_idx(j):
        k = j % 4
        idd[j] = (
            pltpu.async_copy(ei_hbm.at[0, pl.ds(base + j * CH, CH)],
                             sidx_v.at[k], isem.at[k]),
            pltpu.async_copy(ei_hbm.at[1, pl.ds(base + j * CH, CH)],
                             didx_v.at[k], isem.at[k]),
        )

    def fire_gather(j):
        k = j % 4
        idd[j][0].wait()
        idd[j][1].wait()
        gd[j] = pltpu.async_copy(g_hbm.at[sidx_v.at[k]], buf_v.at[k],
                                 gsem.at[k])

    def fire_scatter(j):
        k = j % 4
        gd[j].wait()
        sd[j] = pltpu.async_copy(buf_v.at[k], acc_sh.at[didx_v.at[k]],
                                 ssem.at[j % 2], add=True)

    fire_idx(0)
    fire_idx(1)
    zd = pltpu.async_copy(z_v, acc_sh.at[pl.ds(sid * RPT, RPT)], gsem.at[3])
    fire_gather(0)
    zd.wait()
    plsc.subcore_barrier()
    for j in range(NFC):
        if j + 2 < NFC:
            if j + 2 >= 4:
                sd[j - 2].wait()
            fire_idx(j + 2)
        if j + 1 < NFC:
            fire_gather(j + 1)
        fire_scatter(j)
    for j in range(NFC - 4, NFC):
        sd[j].wait()
    plsc.subcore_barrier()
    pltpu.sync_copy(acc_sh.at[pl.ds(sid * RPT, RPT)],
                    out_hbm.at[cid, pl.ds(sid * RPT, RPT)])


NPK = NP // 8
NK = N // 8


def _tc_a_body(xg_ref, w1e_ref, d2_ref, g1_ref):
    p = jnp.dot(xg_ref[...], w1e_ref[...], preferred_element_type=jnp.float32)
    g1_ref[...] = p * d2_ref[:NK, :]


_tc_a = pl.pallas_call(
    _tc_a_body,
    out_shape=jax.ShapeDtypeStruct((NK, 128), jnp.float32),
)


def _tc_b_body(s_ref, g1_ref, d2_ref, b1_ref, w2_ref, g2_ref):
    s = s_ref[0, :NK, :] + s_ref[1, :NK, :]
    d2 = d2_ref[:NK, :]
    h1 = jnp.maximum(d2 * (s + g1_ref[...]) + b1_ref[...], 0.0)
    g2_ref[...] = jnp.dot(h1, w2_ref[...],
                          preferred_element_type=jnp.float32) * d2


_tc_b = pl.pallas_call(
    _tc_b_body,
    out_shape=jax.ShapeDtypeStruct((NK, 128), jnp.float32),
)


def _tc_c_body(s_ref, g2_ref, d2_ref, b2_ref, out_ref):
    s = s_ref[0, :NK, :] + s_ref[1, :NK, :]
    out_ref[...] = d2_ref[:NK, :] * s + g2_ref[...] * d2_ref[:NK, :] \
        + b2_ref[...]


_tc_c = pl.pallas_call(
    _tc_c_body,
    out_shape=jax.ShapeDtypeStruct((NK, 128), jnp.float32),
)


def kernel(x, edge_index, W1, b1, W2, b2):
    ei = edge_index.astype(jnp.int32)
    eye8 = jnp.eye(8, dtype=jnp.float32)
    w1e = jnp.kron(eye8, W1)
    w2bd = jnp.kron(eye8, jnp.pad(W2, ((0, 0), (0, H1 - H2))))
    b1t = jnp.tile(b1, 8)[None, :]
    b2t = jnp.tile(jnp.pad(b2, (0, H1 - H2)), 8)[None, :]

    d2p = _deg_dinv(ei).reshape(NPK, 128)
    xg = x.reshape(NK, 8 * 128)
    g1p = _tc_a(xg, w1e, d2p)
    s1 = _edge_scatter(g1p.reshape(N, H1), ei)
    g2p = _tc_b(s1.reshape(NC, NPK, 128), g1p, d2p, b1t, w2bd)
    s2 = _edge_scatter(g2p.reshape(N, H1), ei)
    outp = _tc_c(s2.reshape(NC, NPK, 128), g2p, d2p, b2t)
    return outp.reshape(N, H1)[:, :H2]
